# baseline (speedup 1.0000x reference)
import jax
import jax.numpy as jnp
from jax import lax
from jax.experimental import pallas as pl
from jax.experimental.pallas import tpu as pltpu
from jax.experimental.pallas import tpu_sc as plsc

N = 10000
E = 320000
D = 128
DE = 16
H1 = 256
H2 = 32

NC = 2
NS = 16
NW = NC * NS
L = 16

_mesh = lambda: plsc.VectorSubcoreMesh(core_axis_name="c", subcore_axis_name="s")
_params = lambda: pltpu.CompilerParams(needs_layout_passes=False)

f32 = jnp.float32
i32 = jnp.int32


def _t1a_body(x_ref, wl_ref, bl_ref, wr_ref, br_ref, xl_ref, xr_ref, xls_ref):
    xb = x_ref[...]
    xlv = jnp.dot(xb, wl_ref[...], preferred_element_type=f32) + bl_ref[...][None, :]
    xl_ref[...] = xlv
    xls_ref[...] = xlv
    xr_ref[...] = (
        jnp.dot(xb, wr_ref[...], preferred_element_type=f32) + br_ref[...][None, :]
    )


def _dense_nodes1(x, Wl1, bl1, Wr1, br1):
    B = 1000
    nb = N // B
    return pl.pallas_call(
        _t1a_body,
        grid=(nb, 2),
        in_specs=[
            pl.BlockSpec((B, D), lambda i, h: (i, 0)),
            pl.BlockSpec((D, 128), lambda i, h: (0, h)),
            pl.BlockSpec((128,), lambda i, h: (h,)),
            pl.BlockSpec((D, 128), lambda i, h: (0, h)),
            pl.BlockSpec((128,), lambda i, h: (h,)),
        ],
        out_specs=[
            pl.BlockSpec((B, 128), lambda i, h: (i, h)),
            pl.BlockSpec((B, 128), lambda i, h: (i, h)),
            pl.BlockSpec((B, 128), lambda i, h: (h * nb + i, 0)),
        ],
        out_shape=[
            jax.ShapeDtypeStruct((N, H1), f32),
            jax.ShapeDtypeStruct((N, H1), f32),
            jax.ShapeDtypeStruct((2 * N, 128), f32),
        ],
    )(x, Wl1, bl1, Wr1, br1)


def _t1b_body(ea_ref, we1_ref, we2_ref, e1_ref, e2_ref):
    ea = ea_ref[...]
    e1_ref[...] = jnp.dot(ea, we1_ref[...], preferred_element_type=f32)
    e2_ref[...] = jnp.dot(ea, we2_ref[...], preferred_element_type=f32)


def _dense_edges(edge_attr, We1, We2):
    BE = 4000
    return pl.pallas_call(
        _t1b_body,
        grid=(E // BE,),
        in_specs=[
            pl.BlockSpec((BE, DE), lambda i: (i, 0)),
            pl.BlockSpec((DE, H1), lambda i: (0, 0)),
            pl.BlockSpec((DE, H2), lambda i: (0, 0)),
        ],
        out_specs=[
            pl.BlockSpec((BE, H1), lambda i: (i, 0)),
            pl.BlockSpec((BE, H2), lambda i: (i, 0)),
        ],
        out_shape=[
            jax.ShapeDtypeStruct((E, H1), f32),
            jax.ShapeDtypeStruct((E, H2), f32),
        ],
    )(edge_attr, We1, We2)


def _sum0_body(p_ref, o_ref):
    o_ref[...] = jnp.sum(p_ref[...], axis=0)


def _sum_partials(parts):
    return pl.pallas_call(
        _sum0_body,
        out_shape=jax.ShapeDtypeStruct((parts.shape[1],), f32),
    )(parts)


def _t3_body(lo_ref, hi_ref, b1_ref, wl_ref, bl_ref, wr_ref, br_ref,
             xl2_ref, xr2_ref):
    h1 = jnp.concatenate(
        [lo_ref[...] + b1_ref[...][None, :128],
         hi_ref[...] + b1_ref[...][None, 128:]], axis=1)
    pad = jnp.zeros((N, 128 - H2), f32)
    xl2 = jnp.dot(h1, wl_ref[...], preferred_element_type=f32) + bl_ref[...][None, :]
    xr2 = jnp.dot(h1, wr_ref[...], preferred_element_type=f32) + br_ref[...][None, :]
    xl2_ref[...] = jnp.concatenate([xl2, pad], axis=1)
    xr2_ref[...] = jnp.concatenate([xr2, pad], axis=1)


def _dense_nodes2(acc, bias1, Wl2, bl2, Wr2, br2):
    return pl.pallas_call(
        _t3_body,
        grid=(1,),
        in_specs=[
            pl.BlockSpec((N, 128), lambda i: (0, 0)),
            pl.BlockSpec((N, 128), lambda i: (1, 0)),
            pl.BlockSpec((H1,), lambda i: (0,)),
            pl.BlockSpec((H1, H2), lambda i: (0, 0)),
            pl.BlockSpec((H2,), lambda i: (0,)),
            pl.BlockSpec((H1, H2), lambda i: (0, 0)),
            pl.BlockSpec((H2,), lambda i: (0,)),
        ],
        out_specs=[
            pl.BlockSpec((N, 128), lambda i: (0, 0)),
            pl.BlockSpec((N, 128), lambda i: (0, 0)),
        ],
        out_shape=[
            jax.ShapeDtypeStruct((N, 128), f32),
            jax.ShapeDtypeStruct((N, 128), f32),
        ],
    )(acc, acc, bias1, Wl2, bl2, Wr2, br2)


def _t5_body(p_ref, xl_ref, b_ref, o_ref):
    w = jnp.sum(p_ref[...], axis=0)
    v = jnp.dot(w[None, :], xl_ref[...], preferred_element_type=f32)[0]
    o_ref[...] = v[:H2] * (1.0 / N) + b_ref[...]


def _finalize(parts, xl2, bias2):
    return pl.pallas_call(
        _t5_body,
        out_shape=jax.ShapeDtypeStruct((H2,), f32),
    )(parts, xl2, bias2)


_GDN = lax.GatherDimensionNumbers(
    offset_dims=(), collapsed_slice_dims=(0,), start_index_map=(0,))


def _lane_perm(v, idx):
    return lax.gather(
        v, idx[:, None], _GDN, slice_sizes=(1,),
        mode=lax.GatherScatterMode.PROMISE_IN_BOUNDS)


def _hsum(v):
    lanes = lax.iota(i32, L)
    for sh in (8, 4, 2, 1):
        v = v + _lane_perm(v, (lanes + sh) & (L - 1))
    return v


def _zero_vmem_1d(ref, n):
    def body(i, _):
        ref[pl.ds(i * L, L)] = jnp.zeros((L,), f32)
        return 0
    lax.fori_loop(0, n // L, body, 0)


def _seg_accum(tile_ref, idx16, val16):
    lanes = lax.iota(i32, L)
    for jj in range(L):
        mjj = lanes == jj
        cur = plsc.load_gather(tile_ref, [idx16], mask=mjj)
        plsc.store_scatter(tile_ref, [idx16], cur + val16, mask=mjj)


def _sc_logits(xl, xr, e, src, dst, att, width, CC):
    FB = width // L
    tcols = xl.shape[1]
    NCH = E // CC
    trips = -(-NCH // NW)
    slots = trips + (trips % 2)

    def body(xl_hbm, xr_hbm, e_hbm, src_hbm, dst_hbm, att_hbm,
             ex_out, dpart_out,
             srcv0, srcv1, dstv0, dstv1, xa0, xa1, xb0, xb1, eb0, eb1,
             exbuf, attv, denom_tile, sem0, sem1):
        srcvs = (srcv0, srcv1)
        dstvs = (dstv0, dstv1)
        xas = (xa0, xa1)
        xbs = (xb0, xb1)
        ebs = (eb0, eb1)
        sems = (sem0, sem1)

        c = lax.axis_index("c")
        s = lax.axis_index("s")
        wid = s * NC + c

        def prefetch(pb, i):
            @pl.when(wid + i * NW < NCH)
            def _():
                base = (wid + i * NW) * CC
                pltpu.sync_copy(src_hbm.at[pl.ds(base, CC)], srcvs[pb])
                pltpu.sync_copy(dst_hbm.at[pl.ds(base, CC)], dstvs[pb])
                pltpu.async_copy(xl_hbm.at[srcvs[pb]], xas[pb], sems[pb])
                pltpu.async_copy(xr_hbm.at[dstvs[pb]], xbs[pb], sems[pb])
                pltpu.async_copy(e_hbm.at[pl.ds(base, CC)], ebs[pb], sems[pb])

        def compute(pb, i):
            @pl.when(wid + i * NW < NCH)
            def _():
                base = (wid + i * NW) * CC
                pltpu.make_async_copy(xl_hbm.at[srcvs[pb]], xas[pb], sems[pb]).wait()
                pltpu.make_async_copy(xr_hbm.at[dstvs[pb]], xbs[pb], sems[pb]).wait()
                pltpu.make_async_copy(e_hbm.at[pl.ds(base, CC)], ebs[pb], sems[pb]).wait()

                attr = [attv[pl.ds(f * L, L)] for f in range(FB)]

                def group(g, _):
                    def edge(j, logits_v):
                        row = g * L + j
                        acc = jnp.zeros((L,), f32)
                        for f in range(FB):
                            sl = pl.ds(f * L, L)
                            m = xas[pb][row, sl] + xbs[pb][row, sl] + ebs[pb][row, sl]
                            m = jnp.where(m > 0, m, m * 0.2)
                            acc = acc + m * attr[f]
                        lanes = lax.iota(i32, L)
                        return jnp.where(lanes == j, _hsum(acc), logits_v)

                    logits_v = lax.fori_loop(0, L, edge, jnp.zeros((L,), f32),
                                             unroll=4)
                    exv = jnp.exp(logits_v)
                    exbuf[pl.ds(g * L, L)] = exv
                    _seg_accum(denom_tile, dstvs[pb][pl.ds(g * L, L)], exv)
                    return 0

                lax.fori_loop(0, CC // L, group, 0)
                pltpu.sync_copy(exbuf, ex_out.at[pl.ds(base, CC)])

        pltpu.sync_copy(att_hbm, attv)
        prefetch(0, 0)
        _zero_vmem_1d(denom_tile, N)

        def pair(k2, _):
            i0 = 2 * k2
            prefetch(1, i0 + 1)
            compute(0, i0)
            prefetch(0, i0 + 2)
            compute(1, i0 + 1)
            return 0

        lax.fori_loop(0, slots // 2, pair, 0)
        pltpu.sync_copy(denom_tile, dpart_out.at[pl.ds(wid * N, N)])

    k = pl.kernel(
        body,
        out_type=[
            jax.ShapeDtypeStruct((E,), f32),
            jax.ShapeDtypeStruct((NW * N,), f32),
        ],
        mesh=_mesh(),
        compiler_params=_params(),
        scratch_types=[
            pltpu.VMEM((CC,), i32),
            pltpu.VMEM((CC,), i32),
            pltpu.VMEM((CC,), i32),
            pltpu.VMEM((CC,), i32),
            pltpu.VMEM((CC, tcols), f32),
            pltpu.VMEM((CC, tcols), f32),
            pltpu.VMEM((CC, tcols), f32),
            pltpu.VMEM((CC, tcols), f32),
            pltpu.VMEM((CC, width), f32),
            pltpu.VMEM((CC, width), f32),
            pltpu.VMEM((CC,), f32),
            pltpu.VMEM((width,), f32),
            pltpu.VMEM((N,), f32),
            pltpu.SemaphoreType.DMA,
            pltpu.SemaphoreType.DMA,
        ],
    )
    return k(xl, xr, e, src, dst, att)


def _sc_aggregate1(xls, src, dst, ex, denom):
    CC = 128
    NCH = E // CC
    trips = -(-NCH // NS)
    slots = trips + (trips % 2)
    ZR = 80
    NQ = N // ZR

    def body(xls_hbm, src_hbm, dst_hbm, ex_hbm, denom_hbm, out_hbm,
             srcv0, srcv1, dstv0, dstv1, rows0, rows1, exv0, exv1,
             alphav, denom_tile, acc_spmem,
             sem0, sem1, ssem0, ssem1):
        zbuf = rows0
        srcvs = (srcv0, srcv1)
        dstvs = (dstv0, dstv1)
        rows = (rows0, rows1)
        exvs = (exv0, exv1)
        sems = (sem0, sem1)
        ssems = (ssem0, ssem1)

        c = lax.axis_index("c")
        s = lax.axis_index("s")
        coff = c * N
        nq = (NQ - s + NS - 1) // NS

        def prefetch(pb, i):
            @pl.when(s + i * NS < NCH)
            def _():
                base = (s + i * NS) * CC
                pltpu.sync_copy(src_hbm.at[pl.ds(base, CC)], srcvs[pb])
                pltpu.sync_copy(dst_hbm.at[pl.ds(base, CC)], dstvs[pb])
                for ii in range(CC // L):
                    sl = pl.ds(ii * L, L)
                    srcvs[pb][sl] = srcvs[pb][sl] + coff
                pltpu.async_copy(xls_hbm.at[srcvs[pb]], rows[pb], sems[pb])
                pltpu.sync_copy(ex_hbm.at[pl.ds(base, CC)], exvs[pb])

        def compute(pb, i):
            @pl.when(s + i * NS < NCH)
            def _():
                pltpu.make_async_copy(
                    xls_hbm.at[srcvs[pb]], rows[pb], sems[pb]).wait()
                def scale_group(g, _):
                    sl = pl.ds(g * L, L)
                    den = plsc.load_gather(denom_tile, [dstvs[pb][sl]])
                    alpha16 = exvs[pb][sl] / (den + 1e-16)
                    for j in range(L):
                        a = _lane_perm(alpha16, jnp.full((L,), j, i32))
                        row = g * L + j
                        for f in range(128 // L):
                            fl = pl.ds(f * L, L)
                            rows[pb][row, fl] = rows[pb][row, fl] * a
                    return 0

                lax.fori_loop(0, CC // L, scale_group, 0, unroll=2)
                pltpu.async_copy(
                    rows[pb], acc_spmem.at[dstvs[pb]], ssems[pb], add=True).wait()

        def zrow(r, _):
            for ii in range(128 // L):
                zbuf[r, pl.ds(ii * L, L)] = jnp.zeros((L,), f32)
            return 0

        lax.fori_loop(0, ZR, zrow, 0)

        def zchunk(ii, _):
            pltpu.sync_copy(zbuf.at[pl.ds(0, ZR)],
                            acc_spmem.at[pl.ds((s + ii * NS) * ZR, ZR)])
            return 0

        lax.fori_loop(0, nq, zchunk, 0)
        pltpu.sync_copy(denom_hbm, denom_tile)
        prefetch(0, 0)
        plsc.subcore_barrier()

        def pair(k2, _):
            i0 = 2 * k2
            prefetch(1, i0 + 1)
            compute(0, i0)
            prefetch(0, i0 + 2)
            compute(1, i0 + 1)
            return 0

        lax.fori_loop(0, slots // 2, pair, 0)
        plsc.subcore_barrier()

        def ochunk(ii, _):
            q = s + ii * NS
            pltpu.sync_copy(acc_spmem.at[pl.ds(q * ZR, ZR)], zbuf.at[pl.ds(0, ZR)])
            pltpu.sync_copy(zbuf.at[pl.ds(0, ZR)], out_hbm.at[pl.ds(coff + q * ZR, ZR)])
            return 0

        lax.fori_loop(0, nq, ochunk, 0)

    k = pl.kernel(
        body,
        out_type=jax.ShapeDtypeStruct((2 * N, 128), f32),
        mesh=_mesh(),
        compiler_params=_params(),
        scratch_types=[
            pltpu.VMEM((CC,), i32),
            pltpu.VMEM((CC,), i32),
            pltpu.VMEM((CC,), i32),
            pltpu.VMEM((CC,), i32),
            pltpu.VMEM((CC, 128), f32),
            pltpu.VMEM((CC, 128), f32),
            pltpu.VMEM((CC,), f32),
            pltpu.VMEM((CC,), f32),
            pltpu.VMEM((CC + L,), f32),
            pltpu.VMEM((N,), f32),
            pltpu.VMEM_SHARED((N, 128), f32),
            pltpu.SemaphoreType.DMA,
            pltpu.SemaphoreType.DMA,
            pltpu.SemaphoreType.DMA,
            pltpu.SemaphoreType.DMA,
        ],
    )
    return k(xls, src, dst, ex, denom)


def _sc_final(src, dst, ex2, denom2):
    CC = 400
    EPW = E // NW

    def body(src_hbm, dst_hbm, ex_hbm, denom_hbm, part_out,
             srcv, dstv, exv, denom_tile, w_tile, sem):
        c = lax.axis_index("c")
        s = lax.axis_index("s")
        wid = s * NC + c
        base0 = wid * EPW

        pltpu.sync_copy(denom_hbm, denom_tile)
        _zero_vmem_1d(w_tile, N)

        def chunk(k, _):
            base = base0 + k * CC
            pltpu.sync_copy(src_hbm.at[pl.ds(base, CC)], srcv)
            pltpu.sync_copy(dst_hbm.at[pl.ds(base, CC)], dstv)
            pltpu.sync_copy(ex_hbm.at[pl.ds(base, CC)], exv)

            def group(g, _):
                sl = pl.ds(g * L, L)
                den = plsc.load_gather(denom_tile, [dstv[sl]])
                alpha16 = exv[sl] / (den + 1e-16)
                _seg_accum(w_tile, srcv[sl], alpha16)
                return 0

            lax.fori_loop(0, CC // L, group, 0)
            return 0

        lax.fori_loop(0, EPW // CC, chunk, 0)
        pltpu.sync_copy(w_tile, part_out.at[pl.ds(wid * N, N)])

    k = pl.kernel(
        body,
        out_type=jax.ShapeDtypeStruct((NW * N,), f32),
        mesh=_mesh(),
        compiler_params=_params(),
        scratch_types=[
            pltpu.VMEM((CC,), i32),
            pltpu.VMEM((CC,), i32),
            pltpu.VMEM((CC,), f32),
            pltpu.VMEM((N,), f32),
            pltpu.VMEM((N,), f32),
            pltpu.SemaphoreType.DMA,
        ],
    )
    return k(src, dst, ex2, denom2)


def kernel(x, edge_index, edge_attr,
           Wl1, bl1, Wr1, br1, We1, att1, bias1,
           Wl2, bl2, Wr2, br2, We2, att2, bias2):
    src = edge_index[0]
    dst = edge_index[1]

    xl1, xr1, xls = _dense_nodes1(x, Wl1, bl1, Wr1, br1)
    e1, e2 = _dense_edges(edge_attr, We1, We2)

    ex1, dparts1 = _sc_logits(xl1, xr1, e1, src, dst, att1, H1, 64)
    denom1 = _sum_partials(dparts1.reshape(NW, N))
    acc = _sc_aggregate1(xls, src, dst, ex1, denom1)

    xl2, xr2 = _dense_nodes2(acc, bias1, Wl2, bl2, Wr2, br2)
    ex2, dparts2 = _sc_logits(xl2, xr2, e2, src, dst, att2, H2, 128)
    denom2 = _sum_partials(dparts2.reshape(NW, N))
    wparts = _sc_final(src, dst, ex2, denom2)

    return _finalize(wparts.reshape(NW, N), xl2, bias2)


# speedup vs baseline: 1.8426x; 1.8426x over previous
import jax
import jax.numpy as jnp
import numpy as np
from jax import lax
from jax.experimental import pallas as pl
from jax.experimental.pallas import tpu as pltpu
from jax.experimental.pallas import tpu_sc as plsc

N = 10000
E = 320000
D = 128
DE = 16
H1 = 256
H2 = 32

NC = 2
NS = 16
NW = NC * NS
L = 16

_mesh = lambda: plsc.VectorSubcoreMesh(core_axis_name="c", subcore_axis_name="s")
_params = lambda: pltpu.CompilerParams(needs_layout_passes=False)

f32 = jnp.float32
i32 = jnp.int32


u32 = jnp.uint32


def _pack_bf16(a, b):
    ba = lax.bitcast_convert_type(a, u32)
    bb = lax.bitcast_convert_type(b, u32)
    ra = (ba + 0x7FFF + ((ba >> 16) & 1)) >> 16
    rb = (bb + 0x7FFF + ((bb >> 16) & 1)) >> 16
    return lax.bitcast_convert_type(ra | (rb << 16), jnp.int32)


def _t1a_body(x_ref, wl_ref, bl_ref, xls_ref):
    xb = x_ref[...]
    xls_ref[...] = (
        jnp.dot(xb, wl_ref[...], preferred_element_type=f32) + bl_ref[...][None, :]
    )


def _dense_split(x, Wl1, bl1):
    B = 1000
    nb = N // B
    return pl.pallas_call(
        _t1a_body,
        grid=(nb, 2),
        in_specs=[
            pl.BlockSpec((B, D), lambda i, h: (i, 0)),
            pl.BlockSpec((D, 128), lambda i, h: (0, h)),
            pl.BlockSpec((128,), lambda i, h: (h,)),
        ],
        out_specs=pl.BlockSpec((B, 128), lambda i, h: (h * nb + i, 0)),
        out_shape=jax.ShapeDtypeStruct((2 * N, 128), f32),
    )(x, Wl1, bl1)


def _t1p_body(x_ref, wla_ref, bla_ref, wlb_ref, blb_ref,
              wra_ref, bra_ref, wrb_ref, brb_ref, xl_ref, xr_ref):
    xb = x_ref[...]
    xla = jnp.dot(xb, wla_ref[...], preferred_element_type=f32) + bla_ref[...][None, :]
    xlb = jnp.dot(xb, wlb_ref[...], preferred_element_type=f32) + blb_ref[...][None, :]
    xra = jnp.dot(xb, wra_ref[...], preferred_element_type=f32) + bra_ref[...][None, :]
    xrb = jnp.dot(xb, wrb_ref[...], preferred_element_type=f32) + brb_ref[...][None, :]
    xl_ref[...] = _pack_bf16(xla, xlb)
    xr_ref[...] = _pack_bf16(xra, xrb)


def _dense_packed(x, WlA, blA, WlB, blB, WrA, brA, WrB, brB):
    B = 1000
    nb = N // B
    w = lambda: pl.BlockSpec((D, 128), lambda i: (0, 0))
    v = lambda: pl.BlockSpec((128,), lambda i: (0,))
    return pl.pallas_call(
        _t1p_body,
        grid=(nb,),
        in_specs=[
            pl.BlockSpec((B, D), lambda i: (i, 0)),
            w(), v(), w(), v(), w(), v(), w(), v(),
        ],
        out_specs=[
            pl.BlockSpec((B, 128), lambda i: (i, 0)),
            pl.BlockSpec((B, 128), lambda i: (i, 0)),
        ],
        out_shape=[
            jax.ShapeDtypeStruct((N, 128), i32),
            jax.ShapeDtypeStruct((N, 128), i32),
        ],
    )(x, WlA, blA, WlB, blB, WrA, brA, WrB, brB)


def _t1b_body(ea_ref, wea_ref, web_ref, we2_ref, e1_ref, e2_ref):
    ea = ea_ref[...]
    e1a = jnp.dot(ea, wea_ref[...], preferred_element_type=f32)
    e1b = jnp.dot(ea, web_ref[...], preferred_element_type=f32)
    e1_ref[...] = _pack_bf16(e1a, e1b)
    e2_ref[...] = jnp.dot(ea, we2_ref[...], preferred_element_type=f32)


def _dense_edges(edge_attr, WeA, WeB, We2):
    BE = 4000
    return pl.pallas_call(
        _t1b_body,
        grid=(E // BE,),
        in_specs=[
            pl.BlockSpec((BE, DE), lambda i: (i, 0)),
            pl.BlockSpec((DE, 128), lambda i: (0, 0)),
            pl.BlockSpec((DE, 128), lambda i: (0, 0)),
            pl.BlockSpec((DE, H2), lambda i: (0, 0)),
        ],
        out_specs=[
            pl.BlockSpec((BE, 128), lambda i: (i, 0)),
            pl.BlockSpec((BE, H2), lambda i: (i, 0)),
        ],
        out_shape=[
            jax.ShapeDtypeStruct((E, 128), i32),
            jax.ShapeDtypeStruct((E, H2), f32),
        ],
    )(edge_attr, WeA, WeB, We2)


def _sum0_body(p_ref, o_ref):
    o_ref[...] = jnp.sum(p_ref[...], axis=0)


def _sum_partials(parts):
    return pl.pallas_call(
        _sum0_body,
        out_shape=jax.ShapeDtypeStruct((parts.shape[1],), f32),
    )(parts)


def _t3_body(lo_ref, hi_ref, b1_ref, wl_ref, bl_ref, wr_ref, br_ref,
             xl2_ref, xr2_ref):
    h1 = jnp.concatenate(
        [lo_ref[...] + b1_ref[...][None, :128],
         hi_ref[...] + b1_ref[...][None, 128:]], axis=1)
    pad = jnp.zeros((N, 128 - H2), f32)
    xl2 = jnp.dot(h1, wl_ref[...], preferred_element_type=f32) + bl_ref[...][None, :]
    xr2 = jnp.dot(h1, wr_ref[...], preferred_element_type=f32) + br_ref[...][None, :]
    xl2_ref[...] = jnp.concatenate([xl2, pad], axis=1)
    xr2_ref[...] = jnp.concatenate([xr2, pad], axis=1)


def _dense_nodes2(acc, bias1, Wl2, bl2, Wr2, br2):
    return pl.pallas_call(
        _t3_body,
        grid=(1,),
        in_specs=[
            pl.BlockSpec((N, 128), lambda i: (0, 0)),
            pl.BlockSpec((N, 128), lambda i: (1, 0)),
            pl.BlockSpec((H1,), lambda i: (0,)),
            pl.BlockSpec((H1, H2), lambda i: (0, 0)),
            pl.BlockSpec((H2,), lambda i: (0,)),
            pl.BlockSpec((H1, H2), lambda i: (0, 0)),
            pl.BlockSpec((H2,), lambda i: (0,)),
        ],
        out_specs=[
            pl.BlockSpec((N, 128), lambda i: (0, 0)),
            pl.BlockSpec((N, 128), lambda i: (0, 0)),
        ],
        out_shape=[
            jax.ShapeDtypeStruct((N, 128), f32),
            jax.ShapeDtypeStruct((N, 128), f32),
        ],
    )(acc, acc, bias1, Wl2, bl2, Wr2, br2)


def _t5_body(p_ref, xl_ref, b_ref, o_ref):
    w = jnp.sum(p_ref[...], axis=0)
    v = jnp.dot(w[None, :], xl_ref[...], preferred_element_type=f32)[0]
    o_ref[...] = v[:H2] * (1.0 / N) + b_ref[...]


def _finalize(parts, xl2, bias2):
    return pl.pallas_call(
        _t5_body,
        out_shape=jax.ShapeDtypeStruct((H2,), f32),
    )(parts, xl2, bias2)


_GDN = lax.GatherDimensionNumbers(
    offset_dims=(), collapsed_slice_dims=(0,), start_index_map=(0,))


def _lane_perm(v, idx):
    return lax.gather(
        v, idx[:, None], _GDN, slice_sizes=(1,),
        mode=lax.GatherScatterMode.PROMISE_IN_BOUNDS)


def _hsum(v):
    lanes = lax.iota(i32, L)
    for sh in (8, 4, 2, 1):
        v = v + _lane_perm(v, (lanes + sh) & (L - 1))
    return v


def _zero_vmem_1d(ref, n):
    def body(i, _):
        ref[pl.ds(i * L, L)] = jnp.zeros((L,), f32)
        return 0
    lax.fori_loop(0, n // L, body, 0)


def _seg_accum(tile_ref, idx16, val16):
    lanes = lax.iota(i32, L)
    for jj in range(L):
        mjj = lanes == jj
        cur = plsc.load_gather(tile_ref, [idx16], mask=mjj)
        plsc.store_scatter(tile_ref, [idx16], cur + val16, mask=mjj)


def _sc_logits(xl, xr, e, src, dst, att, width, CC):
    FB = width // L
    packed = xl.dtype == jnp.int32
    tcols = xl.shape[1]
    tdt = i32 if packed else f32
    ecols = e.shape[1]
    NCH = E // CC
    trips = -(-NCH // NW)
    slots = trips + (trips % 2)

    def body(xl_hbm, xr_hbm, e_hbm, src_hbm, dst_hbm, att_hbm,
             ex_out, dpart_out,
             srcv0, srcv1, dstv0, dstv1, xa0, xa1, xb0, xb1, eb0, eb1,
             exbuf, attv, denom_tile, sem0, sem1):
        srcvs = (srcv0, srcv1)
        dstvs = (dstv0, dstv1)
        xas = (xa0, xa1)
        xbs = (xb0, xb1)
        ebs = (eb0, eb1)
        sems = (sem0, sem1)

        c = lax.axis_index("c")
        s = lax.axis_index("s")
        wid = s * NC + c

        def prefetch(pb, i):
            @pl.when(wid + i * NW < NCH)
            def _():
                base = (wid + i * NW) * CC
                pltpu.sync_copy(src_hbm.at[pl.ds(base, CC)], srcvs[pb])
                pltpu.sync_copy(dst_hbm.at[pl.ds(base, CC)], dstvs[pb])
                pltpu.async_copy(xl_hbm.at[srcvs[pb]], xas[pb], sems[pb])
                pltpu.async_copy(xr_hbm.at[dstvs[pb]], xbs[pb], sems[pb])
                pltpu.async_copy(e_hbm.at[pl.ds(base, CC)], ebs[pb], sems[pb])

        def compute(pb, i):
            @pl.when(wid + i * NW < NCH)
            def _():
                base = (wid + i * NW) * CC
                pltpu.make_async_copy(xl_hbm.at[srcvs[pb]], xas[pb], sems[pb]).wait()
                pltpu.make_async_copy(xr_hbm.at[dstvs[pb]], xbs[pb], sems[pb]).wait()
                pltpu.make_async_copy(e_hbm.at[pl.ds(base, CC)], ebs[pb], sems[pb]).wait()

                attr = [attv[pl.ds(f * L, L)] for f in range(FB)]

                def group(g, _):
                    def edge(j, logits_v):
                        row = g * L + j
                        acc = jnp.zeros((L,), f32)
                        if packed:
                            fmt = plsc.PackFormat.INTERLEAVED
                            for g2 in range(width // (2 * L)):
                                sl = pl.ds(g2 * L, L)
                                alo, ahi = plsc.unpack(
                                    plsc.bitcast(xas[pb][row, sl], jnp.bfloat16),
                                    format=fmt)
                                blo, bhi = plsc.unpack(
                                    plsc.bitcast(xbs[pb][row, sl], jnp.bfloat16),
                                    format=fmt)
                                elo, ehi = plsc.unpack(
                                    plsc.bitcast(ebs[pb][row, sl], jnp.bfloat16),
                                    format=fmt)
                                m = alo + blo + elo
                                m = jnp.where(m > 0, m, m * 0.2)
                                acc = acc + m * attr[2 * g2]
                                m = ahi + bhi + ehi
                                m = jnp.where(m > 0, m, m * 0.2)
                                acc = acc + m * attr[2 * g2 + 1]
                        else:
                            for f in range(FB):
                                sl = pl.ds(f * L, L)
                                m = xas[pb][row, sl] + xbs[pb][row, sl] + ebs[pb][row, sl]
                                m = jnp.where(m > 0, m, m * 0.2)
                                acc = acc + m * attr[f]
                        lanes = lax.iota(i32, L)
                        return jnp.where(lanes == j, _hsum(acc), logits_v)

                    logits_v = lax.fori_loop(0, L, edge, jnp.zeros((L,), f32))
                    exv = jnp.exp(logits_v)
                    exbuf[pl.ds(g * L, L)] = exv
                    _seg_accum(denom_tile, dstvs[pb][pl.ds(g * L, L)], exv)
                    return 0

                lax.fori_loop(0, CC // L, group, 0)
                pltpu.sync_copy(exbuf, ex_out.at[pl.ds(base, CC)])

        pltpu.sync_copy(att_hbm, attv)
        prefetch(0, 0)
        _zero_vmem_1d(denom_tile, N)

        def pair(k2, _):
            i0 = 2 * k2
            prefetch(1, i0 + 1)
            compute(0, i0)
            prefetch(0, i0 + 2)
            compute(1, i0 + 1)
            return 0

        lax.fori_loop(0, slots // 2, pair, 0)
        pltpu.sync_copy(denom_tile, dpart_out.at[pl.ds(wid * N, N)])

    k = pl.kernel(
        body,
        out_type=[
            jax.ShapeDtypeStruct((E,), f32),
            jax.ShapeDtypeStruct((NW * N,), f32),
        ],
        mesh=_mesh(),
        compiler_params=_params(),
        scratch_types=[
            pltpu.VMEM((CC,), i32),
            pltpu.VMEM((CC,), i32),
            pltpu.VMEM((CC,), i32),
            pltpu.VMEM((CC,), i32),
            pltpu.VMEM((CC, tcols), tdt),
            pltpu.VMEM((CC, tcols), tdt),
            pltpu.VMEM((CC, tcols), tdt),
            pltpu.VMEM((CC, tcols), tdt),
            pltpu.VMEM((CC, ecols), tdt),
            pltpu.VMEM((CC, ecols), tdt),
            pltpu.VMEM((CC,), f32),
            pltpu.VMEM((width,), f32),
            pltpu.VMEM((N,), f32),
            pltpu.SemaphoreType.DMA,
            pltpu.SemaphoreType.DMA,
        ],
    )
    return k(xl, xr, e, src, dst, att)


def _sc_aggregate1(xls, src, dst, ex, denom):
    CC = 128
    NCH = E // CC
    trips = -(-NCH // NS)
    slots = trips + (trips % 2)
    ZR = 80
    NQ = N // ZR

    def body(xls_hbm, src_hbm, dst_hbm, ex_hbm, denom_hbm, out_hbm,
             srcv0, srcv1, dstv0, dstv1, rows0, rows1, exv0, exv1,
             alphav, denom_tile, acc_spmem,
             sem0, sem1, ssem0, ssem1):
        zbuf = rows0
        srcvs = (srcv0, srcv1)
        dstvs = (dstv0, dstv1)
        rows = (rows0, rows1)
        exvs = (exv0, exv1)
        sems = (sem0, sem1)
        ssems = (ssem0, ssem1)

        c = lax.axis_index("c")
        s = lax.axis_index("s")
        coff = c * N
        nq = (NQ - s + NS - 1) // NS

        def prefetch(pb, i):
            @pl.when(s + i * NS < NCH)
            def _():
                base = (s + i * NS) * CC
                pltpu.sync_copy(src_hbm.at[pl.ds(base, CC)], srcvs[pb])
                pltpu.sync_copy(dst_hbm.at[pl.ds(base, CC)], dstvs[pb])
                for ii in range(CC // L):
                    sl = pl.ds(ii * L, L)
                    srcvs[pb][sl] = srcvs[pb][sl] + coff
                pltpu.async_copy(xls_hbm.at[srcvs[pb]], rows[pb], sems[pb])
                pltpu.sync_copy(ex_hbm.at[pl.ds(base, CC)], exvs[pb])

        def compute(pb, i):
            @pl.when(s + i * NS < NCH)
            def _():
                pltpu.make_async_copy(
                    xls_hbm.at[srcvs[pb]], rows[pb], sems[pb]).wait()
                def scale_group(g, _):
                    sl = pl.ds(g * L, L)
                    den = plsc.load_gather(denom_tile, [dstvs[pb][sl]])
                    alpha16 = exvs[pb][sl] / (den + 1e-16)
                    for j in range(L):
                        a = _lane_perm(alpha16, jnp.full((L,), j, i32))
                        row = g * L + j
                        for f in range(128 // L):
                            fl = pl.ds(f * L, L)
                            rows[pb][row, fl] = rows[pb][row, fl] * a
                    return 0

                lax.fori_loop(0, CC // L, scale_group, 0)
                pltpu.async_copy(
                    rows[pb], acc_spmem.at[dstvs[pb]], ssems[pb], add=True).wait()

        def zrow(r, _):
            for ii in range(128 // L):
                zbuf[r, pl.ds(ii * L, L)] = jnp.zeros((L,), f32)
            return 0

        lax.fori_loop(0, ZR, zrow, 0)

        def zchunk(ii, _):
            pltpu.sync_copy(zbuf.at[pl.ds(0, ZR)],
                            acc_spmem.at[pl.ds((s + ii * NS) * ZR, ZR)])
            return 0

        lax.fori_loop(0, nq, zchunk, 0)
        pltpu.sync_copy(denom_hbm, denom_tile)
        prefetch(0, 0)
        plsc.subcore_barrier()

        def pair(k2, _):
            i0 = 2 * k2
            prefetch(1, i0 + 1)
            compute(0, i0)
            prefetch(0, i0 + 2)
            compute(1, i0 + 1)
            return 0

        lax.fori_loop(0, slots // 2, pair, 0)
        plsc.subcore_barrier()

        def ochunk(ii, _):
            q = s + ii * NS
            pltpu.sync_copy(acc_spmem.at[pl.ds(q * ZR, ZR)], zbuf.at[pl.ds(0, ZR)])
            pltpu.sync_copy(zbuf.at[pl.ds(0, ZR)], out_hbm.at[pl.ds(coff + q * ZR, ZR)])
            return 0

        lax.fori_loop(0, nq, ochunk, 0)

    k = pl.kernel(
        body,
        out_type=jax.ShapeDtypeStruct((2 * N, 128), f32),
        mesh=_mesh(),
        compiler_params=_params(),
        scratch_types=[
            pltpu.VMEM((CC,), i32),
            pltpu.VMEM((CC,), i32),
            pltpu.VMEM((CC,), i32),
            pltpu.VMEM((CC,), i32),
            pltpu.VMEM((CC, 128), f32),
            pltpu.VMEM((CC, 128), f32),
            pltpu.VMEM((CC,), f32),
            pltpu.VMEM((CC,), f32),
            pltpu.VMEM((CC + L,), f32),
            pltpu.VMEM((N,), f32),
            pltpu.VMEM_SHARED((N, 128), f32),
            pltpu.SemaphoreType.DMA,
            pltpu.SemaphoreType.DMA,
            pltpu.SemaphoreType.DMA,
            pltpu.SemaphoreType.DMA,
        ],
    )
    return k(xls, src, dst, ex, denom)


def _sc_final(src, dst, ex2, denom2):
    CC = 400
    EPW = E // NW

    def body(src_hbm, dst_hbm, ex_hbm, denom_hbm, part_out,
             srcv, dstv, exv, denom_tile, w_tile, sem):
        c = lax.axis_index("c")
        s = lax.axis_index("s")
        wid = s * NC + c
        base0 = wid * EPW

        pltpu.sync_copy(denom_hbm, denom_tile)
        _zero_vmem_1d(w_tile, N)

        def chunk(k, _):
            base = base0 + k * CC
            pltpu.sync_copy(src_hbm.at[pl.ds(base, CC)], srcv)
            pltpu.sync_copy(dst_hbm.at[pl.ds(base, CC)], dstv)
            pltpu.sync_copy(ex_hbm.at[pl.ds(base, CC)], exv)

            def group(g, _):
                sl = pl.ds(g * L, L)
                den = plsc.load_gather(denom_tile, [dstv[sl]])
                alpha16 = exv[sl] / (den + 1e-16)
                _seg_accum(w_tile, srcv[sl], alpha16)
                return 0

            lax.fori_loop(0, CC // L, group, 0)
            return 0

        lax.fori_loop(0, EPW // CC, chunk, 0)
        pltpu.sync_copy(w_tile, part_out.at[pl.ds(wid * N, N)])

    k = pl.kernel(
        body,
        out_type=jax.ShapeDtypeStruct((NW * N,), f32),
        mesh=_mesh(),
        compiler_params=_params(),
        scratch_types=[
            pltpu.VMEM((CC,), i32),
            pltpu.VMEM((CC,), i32),
            pltpu.VMEM((CC,), f32),
            pltpu.VMEM((N,), f32),
            pltpu.VMEM((N,), f32),
            pltpu.SemaphoreType.DMA,
        ],
    )
    return k(src, dst, ex2, denom2)


def kernel(x, edge_index, edge_attr,
           Wl1, bl1, Wr1, br1, We1, att1, bias1,
           Wl2, bl2, Wr2, br2, We2, att2, bias2):
    src = edge_index[0]
    dst = edge_index[1]

    sel = np.arange(H1).reshape(H1 // 32, 2, 16)
    pa, pb_ = sel[:, 0].reshape(-1), sel[:, 1].reshape(-1)

    xls = _dense_split(x, Wl1, bl1)
    xl1, xr1 = _dense_packed(
        x, Wl1[:, pa], bl1[pa], Wl1[:, pb_], bl1[pb_],
        Wr1[:, pa], br1[pa], Wr1[:, pb_], br1[pb_])
    e1, e2 = _dense_edges(edge_attr, We1[:, pa], We1[:, pb_], We2)

    ex1, dparts1 = _sc_logits(xl1, xr1, e1, src, dst, att1, H1, 64)
    denom1 = _sum_partials(dparts1.reshape(NW, N))
    acc = _sc_aggregate1(xls, src, dst, ex1, denom1)

    xl2, xr2 = _dense_nodes2(acc, bias1, Wl2, bl2, Wr2, br2)
    ex2, dparts2 = _sc_logits(xl2, xr2, e2, src, dst, att2, H2, 128)
    denom2 = _sum_partials(dparts2.reshape(NW, N))
    wparts = _sc_final(src, dst, ex2, denom2)

    return _finalize(wparts.reshape(NW, N), xl2, bias2)


# speedup vs baseline: 1.9899x; 1.0799x over previous
import jax
import jax.numpy as jnp
from jax import lax
from jax.experimental import pallas as pl
from jax.experimental.pallas import tpu as pltpu
from jax.experimental.pallas import tpu_sc as plsc

N = 10000
E = 320000
D = 128
DE = 16
H1 = 256
H2 = 32

NC = 2
NS = 16
NW = NC * NS
L = 16

_mesh = lambda: plsc.VectorSubcoreMesh(core_axis_name="c", subcore_axis_name="s")
_params = lambda: pltpu.CompilerParams(needs_layout_passes=False)

f32 = jnp.float32
i32 = jnp.int32


def _t1a_body(x_ref, wl_ref, bl_ref, wr_ref, br_ref, xl_ref, xr_ref, xls_ref):
    xb = x_ref[...]
    xlv = jnp.dot(xb, wl_ref[...], preferred_element_type=f32) + bl_ref[...][None, :]
    xl_ref[...] = xlv
    xls_ref[...] = xlv
    xr_ref[...] = (
        jnp.dot(xb, wr_ref[...], preferred_element_type=f32) + br_ref[...][None, :]
    )


def _dense_nodes1(x, Wl1, bl1, Wr1, br1):
    B = 1000
    nb = N // B
    return pl.pallas_call(
        _t1a_body,
        grid=(nb, 2),
        in_specs=[
            pl.BlockSpec((B, D), lambda i, h: (i, 0)),
            pl.BlockSpec((D, 128), lambda i, h: (0, h)),
            pl.BlockSpec((128,), lambda i, h: (h,)),
            pl.BlockSpec((D, 128), lambda i, h: (0, h)),
            pl.BlockSpec((128,), lambda i, h: (h,)),
        ],
        out_specs=[
            pl.BlockSpec((B, 128), lambda i, h: (i, h)),
            pl.BlockSpec((B, 128), lambda i, h: (i, h)),
            pl.BlockSpec((B, 128), lambda i, h: (h * nb + i, 0)),
        ],
        out_shape=[
            jax.ShapeDtypeStruct((N, H1), f32),
            jax.ShapeDtypeStruct((N, H1), f32),
            jax.ShapeDtypeStruct((2 * N, 128), f32),
        ],
    )(x, Wl1, bl1, Wr1, br1)


def _t1b_body(ea_ref, we1_ref, we2_ref, e1_ref, e2_ref):
    ea = ea_ref[...]
    e1_ref[...] = jnp.dot(ea, we1_ref[...], preferred_element_type=f32)
    e2_ref[...] = jnp.dot(ea, we2_ref[...], preferred_element_type=f32)


def _dense_edges(edge_attr, We1, We2):
    BE = 4000
    return pl.pallas_call(
        _t1b_body,
        grid=(E // BE,),
        in_specs=[
            pl.BlockSpec((BE, DE), lambda i: (i, 0)),
            pl.BlockSpec((DE, H1), lambda i: (0, 0)),
            pl.BlockSpec((DE, H2), lambda i: (0, 0)),
        ],
        out_specs=[
            pl.BlockSpec((BE, H1), lambda i: (i, 0)),
            pl.BlockSpec((BE, H2), lambda i: (i, 0)),
        ],
        out_shape=[
            jax.ShapeDtypeStruct((E, H1), f32),
            jax.ShapeDtypeStruct((E, H2), f32),
        ],
    )(edge_attr, We1, We2)


def _sum0_body(p_ref, o_ref):
    o_ref[...] = jnp.sum(p_ref[...], axis=0)


def _sum_partials(parts):
    return pl.pallas_call(
        _sum0_body,
        out_shape=jax.ShapeDtypeStruct((parts.shape[1],), f32),
    )(parts)


def _t3_body(lo_ref, hi_ref, b1_ref, wl_ref, bl_ref, wr_ref, br_ref,
             xl2_ref, xr2_ref):
    h1 = jnp.concatenate(
        [lo_ref[...] + b1_ref[...][None, :128],
         hi_ref[...] + b1_ref[...][None, 128:]], axis=1)
    pad = jnp.zeros((N, 128 - H2), f32)
    xl2 = jnp.dot(h1, wl_ref[...], preferred_element_type=f32) + bl_ref[...][None, :]
    xr2 = jnp.dot(h1, wr_ref[...], preferred_element_type=f32) + br_ref[...][None, :]
    xl2_ref[...] = jnp.concatenate([xl2, pad], axis=1)
    xr2_ref[...] = jnp.concatenate([xr2, pad], axis=1)


def _dense_nodes2(acc, bias1, Wl2, bl2, Wr2, br2):
    return pl.pallas_call(
        _t3_body,
        grid=(1,),
        in_specs=[
            pl.BlockSpec((N, 128), lambda i: (0, 0)),
            pl.BlockSpec((N, 128), lambda i: (1, 0)),
            pl.BlockSpec((H1,), lambda i: (0,)),
            pl.BlockSpec((H1, H2), lambda i: (0, 0)),
            pl.BlockSpec((H2,), lambda i: (0,)),
            pl.BlockSpec((H1, H2), lambda i: (0, 0)),
            pl.BlockSpec((H2,), lambda i: (0,)),
        ],
        out_specs=[
            pl.BlockSpec((N, 128), lambda i: (0, 0)),
            pl.BlockSpec((N, 128), lambda i: (0, 0)),
        ],
        out_shape=[
            jax.ShapeDtypeStruct((N, 128), f32),
            jax.ShapeDtypeStruct((N, 128), f32),
        ],
    )(acc, acc, bias1, Wl2, bl2, Wr2, br2)


def _t5_body(p_ref, xl_ref, b_ref, o_ref):
    w = jnp.sum(p_ref[...], axis=0)
    v = jnp.dot(w[None, :], xl_ref[...], preferred_element_type=f32)[0]
    o_ref[...] = v[:H2] * (1.0 / N) + b_ref[...]


def _finalize(parts, xl2, bias2):
    return pl.pallas_call(
        _t5_body,
        out_shape=jax.ShapeDtypeStruct((H2,), f32),
    )(parts, xl2, bias2)


_GDN = lax.GatherDimensionNumbers(
    offset_dims=(), collapsed_slice_dims=(0,), start_index_map=(0,))


def _lane_perm(v, idx):
    return lax.gather(
        v, idx[:, None], _GDN, slice_sizes=(1,),
        mode=lax.GatherScatterMode.PROMISE_IN_BOUNDS)


def _hsum(v):
    lanes = lax.iota(i32, L)
    for sh in (8, 4, 2, 1):
        v = v + _lane_perm(v, (lanes + sh) & (L - 1))
    return v


def _zero_vmem_1d(ref, n):
    def body(i, _):
        ref[pl.ds(i * L, L)] = jnp.zeros((L,), f32)
        return 0
    lax.fori_loop(0, n // L, body, 0)


def _seg_accum(tile_ref, idx16, val16):
    lanes = lax.iota(i32, L)
    for jj in range(L):
        mjj = lanes == jj
        cur = plsc.load_gather(tile_ref, [idx16], mask=mjj)
        plsc.store_scatter(tile_ref, [idx16], cur + val16, mask=mjj)


def _sc_logits(xl, xr, e, src, dst, att, width, CC):
    FB = width // L
    tcols = xl.shape[1]
    NCH = E // CC
    trips = -(-NCH // NW)
    slots = trips + (trips % 2)

    def body(xl_hbm, xr_hbm, e_hbm, src_hbm, dst_hbm, att_hbm,
             ex_out, dpart_out,
             srcv0, srcv1, dstv0, dstv1, xa0, xa1, xb0, xb1, eb0, eb1,
             exbuf, attv, denom_tile, sem0, sem1):
        srcvs = (srcv0, srcv1)
        dstvs = (dstv0, dstv1)
        xas = (xa0, xa1)
        xbs = (xb0, xb1)
        ebs = (eb0, eb1)
        sems = (sem0, sem1)

        c = lax.axis_index("c")
        s = lax.axis_index("s")
        wid = s * NC + c

        def prefetch(pb, i):
            @pl.when(wid + i * NW < NCH)
            def _():
                base = (wid + i * NW) * CC
                pltpu.sync_copy(src_hbm.at[pl.ds(base, CC)], srcvs[pb])
                pltpu.sync_copy(dst_hbm.at[pl.ds(base, CC)], dstvs[pb])
                pltpu.async_copy(xl_hbm.at[srcvs[pb]], xas[pb], sems[pb])
                pltpu.async_copy(xr_hbm.at[dstvs[pb]], xbs[pb], sems[pb])
                pltpu.async_copy(e_hbm.at[pl.ds(base, CC)], ebs[pb], sems[pb])

        def compute(pb, i):
            @pl.when(wid + i * NW < NCH)
            def _():
                base = (wid + i * NW) * CC
                pltpu.make_async_copy(xl_hbm.at[srcvs[pb]], xas[pb], sems[pb]).wait()
                pltpu.make_async_copy(xr_hbm.at[dstvs[pb]], xbs[pb], sems[pb]).wait()
                pltpu.make_async_copy(e_hbm.at[pl.ds(base, CC)], ebs[pb], sems[pb]).wait()

                attr = [attv[pl.ds(f * L, L)] for f in range(FB)]

                def group(g, _):
                    def edge(j, logits_v):
                        row = g * L + j
                        acc = jnp.zeros((L,), f32)
                        for f in range(FB):
                            sl = pl.ds(f * L, L)
                            m = xas[pb][row, sl] + xbs[pb][row, sl] + ebs[pb][row, sl]
                            m = jnp.where(m > 0, m, m * 0.2)
                            acc = acc + m * attr[f]
                        lanes = lax.iota(i32, L)
                        return jnp.where(lanes == j, _hsum(acc), logits_v)

                    logits_v = lax.fori_loop(0, L, edge, jnp.zeros((L,), f32))
                    exv = jnp.exp(logits_v)
                    exbuf[pl.ds(g * L, L)] = exv
                    _seg_accum(denom_tile, dstvs[pb][pl.ds(g * L, L)], exv)
                    return 0

                lax.fori_loop(0, CC // L, group, 0)
                pltpu.sync_copy(exbuf, ex_out.at[pl.ds(base, CC)])

        pltpu.sync_copy(att_hbm, attv)
        prefetch(0, 0)
        _zero_vmem_1d(denom_tile, N)

        def pair(k2, _):
            i0 = 2 * k2
            prefetch(1, i0 + 1)
            compute(0, i0)
            prefetch(0, i0 + 2)
            compute(1, i0 + 1)
            return 0

        lax.fori_loop(0, slots // 2, pair, 0)
        pltpu.sync_copy(denom_tile, dpart_out.at[pl.ds(wid * N, N)])

    k = pl.kernel(
        body,
        out_type=[
            jax.ShapeDtypeStruct((E,), f32),
            jax.ShapeDtypeStruct((NW * N,), f32),
        ],
        mesh=_mesh(),
        compiler_params=_params(),
        scratch_types=[
            pltpu.VMEM((CC,), i32),
            pltpu.VMEM((CC,), i32),
            pltpu.VMEM((CC,), i32),
            pltpu.VMEM((CC,), i32),
            pltpu.VMEM((CC, tcols), f32),
            pltpu.VMEM((CC, tcols), f32),
            pltpu.VMEM((CC, tcols), f32),
            pltpu.VMEM((CC, tcols), f32),
            pltpu.VMEM((CC, width), f32),
            pltpu.VMEM((CC, width), f32),
            pltpu.VMEM((CC,), f32),
            pltpu.VMEM((width,), f32),
            pltpu.VMEM((N,), f32),
            pltpu.SemaphoreType.DMA,
            pltpu.SemaphoreType.DMA,
        ],
    )
    return k(xl, xr, e, src, dst, att)


def _sc_aggregate1(xls, src, dst, ex, denom):
    CC = 128
    NCH = E // CC
    trips = -(-NCH // NS)
    slots = trips + (trips % 2)
    ZR = 80
    NQ = N // ZR

    def body(xls_hbm, src_hbm, dst_hbm, ex_hbm, denom_hbm, out_hbm,
             srcv0, srcv1, dstv0, dstv1, rows0, rows1, exv0, exv1,
             alphav, denom_tile, acc_spmem,
             sem0, sem1, ssem0, ssem1):
        zbuf = rows0
        srcvs = (srcv0, srcv1)
        dstvs = (dstv0, dstv1)
        rows = (rows0, rows1)
        exvs = (exv0, exv1)
        sems = (sem0, sem1)
        ssems = (ssem0, ssem1)

        c = lax.axis_index("c")
        s = lax.axis_index("s")
        coff = c * N
        nq = (NQ - s + NS - 1) // NS

        def prefetch(pb, i):
            @pl.when(s + i * NS < NCH)
            def _():
                base = (s + i * NS) * CC
                pltpu.sync_copy(src_hbm.at[pl.ds(base, CC)], srcvs[pb])
                pltpu.sync_copy(dst_hbm.at[pl.ds(base, CC)], dstvs[pb])
                for ii in range(CC // L):
                    sl = pl.ds(ii * L, L)
                    srcvs[pb][sl] = srcvs[pb][sl] + coff
                pltpu.async_copy(xls_hbm.at[srcvs[pb]], rows[pb], sems[pb])
                pltpu.sync_copy(ex_hbm.at[pl.ds(base, CC)], exvs[pb])

        def compute(pb, i):
            @pl.when(s + i * NS < NCH)
            def _():
                pltpu.make_async_copy(
                    xls_hbm.at[srcvs[pb]], rows[pb], sems[pb]).wait()
                def scale_group(g, _):
                    sl = pl.ds(g * L, L)
                    den = plsc.load_gather(denom_tile, [dstvs[pb][sl]])
                    alpha16 = exvs[pb][sl] / (den + 1e-16)
                    for j in range(L):
                        a = _lane_perm(alpha16, jnp.full((L,), j, i32))
                        row = g * L + j
                        for f in range(128 // L):
                            fl = pl.ds(f * L, L)
                            rows[pb][row, fl] = rows[pb][row, fl] * a
                    return 0

                lax.fori_loop(0, CC // L, scale_group, 0)
                pltpu.async_copy(
                    rows[pb], acc_spmem.at[dstvs[pb]], ssems[pb], add=True).wait()

        def zrow(r, _):
            for ii in range(128 // L):
                zbuf[r, pl.ds(ii * L, L)] = jnp.zeros((L,), f32)
            return 0

        lax.fori_loop(0, ZR, zrow, 0)

        def zchunk(ii, _):
            pltpu.sync_copy(zbuf.at[pl.ds(0, ZR)],
                            acc_spmem.at[pl.ds((s + ii * NS) * ZR, ZR)])
            return 0

        lax.fori_loop(0, nq, zchunk, 0)
        pltpu.sync_copy(denom_hbm, denom_tile)
        prefetch(0, 0)
        plsc.subcore_barrier()

        def pair(k2, _):
            i0 = 2 * k2
            prefetch(1, i0 + 1)
            compute(0, i0)
            prefetch(0, i0 + 2)
            compute(1, i0 + 1)
            return 0

        lax.fori_loop(0, slots // 2, pair, 0)
        plsc.subcore_barrier()

        def ochunk(ii, _):
            q = s + ii * NS
            pltpu.sync_copy(acc_spmem.at[pl.ds(q * ZR, ZR)], zbuf.at[pl.ds(0, ZR)])
            pltpu.sync_copy(zbuf.at[pl.ds(0, ZR)], out_hbm.at[pl.ds(coff + q * ZR, ZR)])
            return 0

        lax.fori_loop(0, nq, ochunk, 0)

    k = pl.kernel(
        body,
        out_type=jax.ShapeDtypeStruct((2 * N, 128), f32),
        mesh=_mesh(),
        compiler_params=_params(),
        scratch_types=[
            pltpu.VMEM((CC,), i32),
            pltpu.VMEM((CC,), i32),
            pltpu.VMEM((CC,), i32),
            pltpu.VMEM((CC,), i32),
            pltpu.VMEM((CC, 128), f32),
            pltpu.VMEM((CC, 128), f32),
            pltpu.VMEM((CC,), f32),
            pltpu.VMEM((CC,), f32),
            pltpu.VMEM((CC + L,), f32),
            pltpu.VMEM((N,), f32),
            pltpu.VMEM_SHARED((N, 128), f32),
            pltpu.SemaphoreType.DMA,
            pltpu.SemaphoreType.DMA,
            pltpu.SemaphoreType.DMA,
            pltpu.SemaphoreType.DMA,
        ],
    )
    return k(xls, src, dst, ex, denom)


def _sc_final(src, dst, ex2, denom2):
    CC = 400
    EPW = E // NW

    def body(src_hbm, dst_hbm, ex_hbm, denom_hbm, part_out,
             srcv, dstv, exv, denom_tile, w_tile, sem):
        c = lax.axis_index("c")
        s = lax.axis_index("s")
        wid = s * NC + c
        base0 = wid * EPW

        pltpu.sync_copy(denom_hbm, denom_tile)
        _zero_vmem_1d(w_tile, N)

        def chunk(k, _):
            base = base0 + k * CC
            pltpu.sync_copy(src_hbm.at[pl.ds(base, CC)], srcv)
            pltpu.sync_copy(dst_hbm.at[pl.ds(base, CC)], dstv)
            pltpu.sync_copy(ex_hbm.at[pl.ds(base, CC)], exv)

            def group(g, _):
                sl = pl.ds(g * L, L)
                den = plsc.load_gather(denom_tile, [dstv[sl]])
                alpha16 = exv[sl] / (den + 1e-16)
                _seg_accum(w_tile, srcv[sl], alpha16)
                return 0

            lax.fori_loop(0, CC // L, group, 0)
            return 0

        lax.fori_loop(0, EPW // CC, chunk, 0)
        pltpu.sync_copy(w_tile, part_out.at[pl.ds(wid * N, N)])

    k = pl.kernel(
        body,
        out_type=jax.ShapeDtypeStruct((NW * N,), f32),
        mesh=_mesh(),
        compiler_params=_params(),
        scratch_types=[
            pltpu.VMEM((CC,), i32),
            pltpu.VMEM((CC,), i32),
            pltpu.VMEM((CC,), f32),
            pltpu.VMEM((N,), f32),
            pltpu.VMEM((N,), f32),
            pltpu.SemaphoreType.DMA,
        ],
    )
    return k(src, dst, ex2, denom2)


def kernel(x, edge_index, edge_attr,
           Wl1, bl1, Wr1, br1, We1, att1, bias1,
           Wl2, bl2, Wr2, br2, We2, att2, bias2):
    src = edge_index[0]
    dst = edge_index[1]

    xl1, xr1, xls = _dense_nodes1(x, Wl1, bl1, Wr1, br1)
    e1, e2 = _dense_edges(edge_attr, We1, We2)

    ex1, dparts1 = _sc_logits(xl1, xr1, e1, src, dst, att1, H1, 64)
    denom1 = _sum_partials(dparts1.reshape(NW, N))
    acc = _sc_aggregate1(xls, src, dst, ex1, denom1)

    xl2, xr2 = _dense_nodes2(acc, bias1, Wl2, bl2, Wr2, br2)
    ex2, dparts2 = _sc_logits(xl2, xr2, e2, src, dst, att2, H2, 128)
    denom2 = _sum_partials(dparts2.reshape(NW, N))
    wparts = _sc_final(src, dst, ex2, denom2)

    return _finalize(wparts.reshape(NW, N), xl2, bias2)
